# baseline (speedup 1.0000x reference)
import functools

import jax
import jax.numpy as jnp
import numpy as np
from jax import lax
from jax.experimental import pallas as pl
from jax.experimental.pallas import tpu as pltpu
from jax.experimental.pallas import tpu_sc as plsc

_N = 524288
_NW = 16
_CHUNK = _N // _NW
_VPC = _CHUNK // 16
_SEG = 4096
_NSEG = _CHUNK // _SEG
_UN = 8
_BLK = 16 * _UN
_ROUNDS = 16
_PCAP = 4096

_MINI32 = np.int32(-2147483648)
_MAXI32 = np.int32(0x7FFFFFFF)
_KNEGINF = np.int32(-2139095041)

_C = [np.float32(c) for c in (
    3.5075520e-06, 9.9979246e-01, -4.9697793e-01, 3.1459054e-01,
    -1.8878268e-01, 8.1726812e-02, -1.7208060e-02)]

_ROWS_PER_TILE = 5
_TILE_STRIDE = _ROWS_PER_TILE * 16
_SLOT = _NW * _TILE_STRIDE
_NSLOTS = _ROUNDS + 1


def _softplus16(x):
    w = jnp.exp(-jnp.abs(x))
    p = jnp.full((16,), _C[6], jnp.float32)
    for c in _C[5::-1]:
        p = p * w + c
    return jnp.maximum(x, 0.0) + p


def _keyify(x, t):
    b = lax.bitcast_convert_type(x, jnp.int32)
    key = jnp.where(b >= 0, b, b ^ _MAXI32)
    return jnp.where(t > 0.0, _MINI32, key)


def _unkey16(key):
    b = jnp.where(key >= 0, key, key ^ _MAXI32)
    return lax.bitcast_convert_type(b, jnp.float32)


def _bisect(a, b):
    d = b - a
    return a + lax.shift_right_logical(d, np.int32(1)) + (d & np.int32(1))


def _sc_loss(x, t):
    mesh = plsc.VectorSubcoreMesh(
        core_axis_name="c", subcore_axis_name="s", num_cores=1
    )

    @functools.partial(
        pl.kernel,
        out_type=jax.ShapeDtypeStruct((16,), jnp.float32),
        mesh=mesh,
        compiler_params=pltpu.CompilerParams(needs_layout_passes=False),
        scratch_types=[
            pltpu.VMEM((_CHUNK,), jnp.int32),
            pltpu.VMEM((_CHUNK + _BLK,), jnp.int32),
            pltpu.VMEM((_CHUNK + _BLK,), jnp.int32),
            pltpu.VMEM((_SEG,), jnp.float32),
            pltpu.VMEM((_SEG,), jnp.float32),
            pltpu.VMEM((_SEG,), jnp.float32),
            pltpu.VMEM((_SEG,), jnp.float32),
            pltpu.VMEM((_PCAP,), jnp.float32),
            pltpu.VMEM((_PCAP,), jnp.float32),
            pltpu.VMEM((_TILE_STRIDE,), jnp.int32),
            pltpu.VMEM((_SLOT,), jnp.int32),
            pltpu.VMEM((16,), jnp.float32),
            pltpu.VMEM_SHARED((_NSLOTS * _SLOT,), jnp.int32),
            pltpu.SemaphoreType.DMA,
            pltpu.SemaphoreType.DMA,
            pltpu.SemaphoreType.DMA,
            pltpu.SemaphoreType.DMA,
        ],
    )
    def body(x_hbm, t_hbm, out_hbm, kv, pa, pb, xs0, xs1, ts0, ts1,
             posx, post, stage, rd, outv, shared,
             sx0, sx1, st0, st1):
        s = lax.axis_index("s")
        base = s * _CHUNK
        xbufs, tbufs, xsems, tsems = (xs0, xs1), (ts0, ts1), (sx0, sx1), (st0, st1)

        def seg_start(i):
            buf = i % 2
            cx = pltpu.async_copy(
                x_hbm.at[pl.ds(base + i * _SEG, _SEG)], xbufs[buf], xsems[buf])
            ct = pltpu.async_copy(
                t_hbm.at[pl.ds(base + i * _SEG, _SEG)], tbufs[buf], tsems[buf])
            return cx, ct

        def merge(slot, rows):
            for ri, row in enumerate(rows):
                stage[pl.ds(ri * 16, 16)] = row
            pltpu.sync_copy(
                stage, shared.at[pl.ds(slot * _SLOT + s * _TILE_STRIDE,
                                       _TILE_STRIDE)])
            plsc.subcore_barrier()
            pltpu.sync_copy(shared.at[pl.ds(slot * _SLOT, _SLOT)], rd)
            out = []
            for ri in range(len(rows)):
                tot = rd[pl.ds(ri * 16, 16)]
                for j in range(1, _NW):
                    tot = tot + rd[pl.ds(j * _TILE_STRIDE + ri * 16, 16)]
                out.append(tot)
            return out

        pend = seg_start(0)
        pwp = np.int32(0)
        for seg in range(_NSEG):
            cx, ct = pend
            cx.wait()
            ct.wait()
            if seg + 1 < _NSEG:
                pend = seg_start(seg + 1)
            xv, tv = xbufs[seg % 2], tbufs[seg % 2]

            def p0(i, wp, seg=seg, xv=xv, tv=tv):
                o = i * _BLK
                for j in range(_UN):
                    xj = xv[pl.ds(o + j * 16, 16)]
                    tj = tv[pl.ds(o + j * 16, 16)]
                    kv[pl.ds(seg * _SEG + o + j * 16, 16)] = _keyify(xj, tj)
                    pm = tj > 0.0
                    plsc.store_compressed(posx.at[pl.ds(wp, 16)], xj, mask=pm)
                    plsc.store_compressed(post.at[pl.ds(wp, 16)], tj, mask=pm)
                    wp = wp + plsc.all_reduce_population_count(pm)[0]
                return wp

            pwp = lax.fori_loop(0, _SEG // _BLK, p0, pwp)

        lane = lax.broadcasted_iota(jnp.int32, (16,), 0)

        def pospass(i, acc):
            o = i * 16
            xj = posx[pl.ds(o, 16)]
            tj = post[pl.ds(o, 16)]
            valid = (o + lane) < pwp
            term = _softplus16(xj) - xj * tj
            return acc + jnp.where(valid, term, 0.0)

        nposv = lax.shift_right_logical(pwp + np.int32(15), np.int32(4))
        possum_v = lax.fori_loop(0, nposv, pospass, jnp.zeros((16,), jnp.float32))

        sent = jnp.full((16,), _MINI32, jnp.int32)
        lo = jnp.full((), _MINI32 + np.int32(1), jnp.int32)
        hi = jnp.full((), _MAXI32, jnp.int32)
        above = jnp.full((), np.int32(0), jnp.int32)
        ki = None
        pos_num = None
        sv = np.int32(_CHUNK)
        bufs = [kv, pa, pb]
        src_i = 0

        for r in range(_ROUNDS):
            m = _bisect(lo, hi)
            ml = _bisect(lo, m - np.int32(1))
            mh = _bisect(m, hi)
            src = bufs[src_i]

            def cpass(i, accs, src=src, ml=ml, m=m, mh=mh):
                a0, a1, a2 = accs
                o = i * _BLK
                for j in range(_UN):
                    vec = src[pl.ds(o + j * 16, 16)]
                    a0 = a0 + jnp.where(vec >= ml, 1, 0)
                    a1 = a1 + jnp.where(vec >= m, 1, 0)
                    a2 = a2 + jnp.where(vec >= mh, 1, 0)
                return (a0, a1, a2)

            nblk = lax.shift_right_logical(sv + np.int32(127), np.int32(7))
            z = jnp.zeros((16,), jnp.int32)
            accs = lax.fori_loop(0, nblk, cpass, (z, z, z))

            if r == 0:
                rows = [accs[0], accs[1], accs[2],
                        jnp.full((16,), pwp, jnp.int32),
                        lax.bitcast_convert_type(possum_v, jnp.int32)]
                g = merge(r, rows)
                pos_num = jnp.sum(g[3])
                ki = pos_num * np.int32(3)
                possum = jnp.sum(lax.bitcast_convert_type(g[4], jnp.float32))
            else:
                g = merge(r, [accs[0], accs[1], accs[2]])
            c_ml = above + jnp.sum(g[0])
            c_m = above + jnp.sum(g[1])
            c_mh = above + jnp.sum(g[2])

            okm = c_m >= ki
            okh = c_mh >= ki
            okl = c_ml >= ki
            lo = jnp.where(okm, jnp.where(okh, mh, m), jnp.where(okl, ml, lo))
            hi = jnp.where(okm, jnp.where(okh, hi, mh - np.int32(1)),
                           jnp.where(okl, m - np.int32(1), ml - np.int32(1)))
            above = jnp.where(okm, jnp.where(okh, above, c_mh),
                              jnp.where(okl, c_m, c_ml))

            if r < _ROUNDS - 1:
                dst = bufs[1] if src_i != 1 else bufs[2]

                def ppass(i, wp, src=src, dst=dst, lo=lo, hi=hi):
                    o = i * _BLK
                    for j in range(_UN):
                        vec = src[pl.ds(o + j * 16, 16)]
                        keep = (vec >= lo) & (vec <= hi)
                        plsc.store_compressed(dst.at[pl.ds(wp, 16)], vec, mask=keep)
                        wp = wp + plsc.all_reduce_population_count(keep)[0]
                    return wp

                wp = lax.fori_loop(0, nblk, ppass, np.int32(0))
                for j in range(_UN):
                    dst[pl.ds(wp + j * 16, 16)] = sent
                sv = wp
                src_i = 1 if src_i != 1 else 2

        v = lo

        negbuf = pb

        def npass(i, wp):
            o = i * _BLK
            for j in range(_UN):
                vec = kv[pl.ds(o + j * 16, 16)]
                gm = vec > v
                plsc.store_compressed(negbuf.at[pl.ds(wp, 16)], vec, mask=gm)
                wp = wp + plsc.all_reduce_population_count(gm)[0]
            return wp

        nwp = lax.fori_loop(0, _VPC // _UN, npass, np.int32(0))
        padk = jnp.full((16,), _KNEGINF, jnp.int32)
        for j in range(_UN):
            negbuf[pl.ds(nwp + j * 16, 16)] = padk

        def spass(i, acc):
            xr = _unkey16(negbuf[pl.ds(i * 16, 16)])
            return acc + _softplus16(xr)

        nnegv = lax.shift_right_logical(nwp + np.int32(15), np.int32(4))
        negsum_v = lax.fori_loop(0, nnegv, spass, jnp.zeros((16,), jnp.float32))

        g = merge(_ROUNDS, [jnp.full((16,), nwp, jnp.int32),
                            lax.bitcast_convert_type(negsum_v, jnp.int32)])
        count_gt = jnp.sum(g[0])
        negsum = jnp.sum(lax.bitcast_convert_type(g[1], jnp.float32))

        sp_v = _softplus16(_unkey16(jnp.full((16,), v, jnp.int32)))
        tie = (ki - count_gt).astype(jnp.float32) * sp_v
        total = (pos_num + ki).astype(jnp.float32)
        loss = (possum + negsum + tie) / total

        @pl.when(s == 0)
        def _():
            outv[...] = loss
            pltpu.sync_copy(outv, out_hbm)

    return body(x, t)


def kernel(input, target):
    out = _sc_loss(input, target)
    return out[0]
